# Initial kernel scaffold; baseline (speedup 1.0000x reference)
import jax
import jax.numpy as jnp
from jax.experimental import pallas as pl


def kernel(input, adj_m, weight, bias):
    raise NotImplementedError("write your pallas kernel here")


# speedup vs baseline: 1.0008x; 1.0008x over previous
import jax
import jax.numpy as jnp
from jax.experimental import pallas as pl
from jax.experimental.pallas import tpu as pltpu

_BM = 200


def _support_body(x_ref, w_ref, o_ref):
    o_ref[...] = jnp.dot(
        x_ref[...], w_ref[...], preferred_element_type=jnp.float32
    ).astype(jnp.bfloat16)


def _spmm_body(adj_ref, sup_ref, bias_ref, o_ref):
    a = adj_ref[...].astype(jnp.bfloat16)
    o_ref[...] = (
        jnp.dot(a, sup_ref[...], preferred_element_type=jnp.float32)
        + bias_ref[...]
    )


def kernel(input, adj_m, weight, bias):
    n, d_in = input.shape
    d_out = weight.shape[1]

    support = pl.pallas_call(
        _support_body,
        out_shape=jax.ShapeDtypeStruct((n, d_out), jnp.bfloat16),
    )(input, weight)

    bias2 = bias.reshape(1, d_out)

    out = pl.pallas_call(
        _spmm_body,
        grid=(n // _BM,),
        in_specs=[
            pl.BlockSpec((_BM, n), lambda i: (i, 0)),
            pl.BlockSpec((n, d_out), lambda i: (0, 0)),
            pl.BlockSpec((1, d_out), lambda i: (0, 0)),
        ],
        out_specs=pl.BlockSpec((_BM, d_out), lambda i: (i, 0)),
        out_shape=jax.ShapeDtypeStruct((n, d_out), jnp.float32),
        compiler_params=pltpu.CompilerParams(
            dimension_semantics=("arbitrary",),
        ),
    )(adj_m, support, bias2)
    return out


# speedup vs baseline: 1.0255x; 1.0247x over previous
import jax
import jax.numpy as jnp
from jax.experimental import pallas as pl
from jax.experimental.pallas import tpu as pltpu

_BM = 200


def _body(adj_ref, x_ref, w_ref, bias_ref, o_ref, sup_ref):
    i = pl.program_id(0)

    @pl.when(i == 0)
    def _():
        sup_ref[...] = jnp.dot(
            x_ref[...], w_ref[...], preferred_element_type=jnp.float32
        ).astype(jnp.bfloat16)

    a = adj_ref[...].astype(jnp.bfloat16)
    o_ref[...] = (
        jnp.dot(a, sup_ref[...], preferred_element_type=jnp.float32)
        + bias_ref[...]
    )


def kernel(input, adj_m, weight, bias):
    n, d_in = input.shape
    d_out = weight.shape[1]
    bias2 = bias.reshape(1, d_out)

    out = pl.pallas_call(
        _body,
        grid=(n // _BM,),
        in_specs=[
            pl.BlockSpec((_BM, n), lambda i: (i, 0)),
            pl.BlockSpec((n, d_in), lambda i: (0, 0)),
            pl.BlockSpec((d_in, d_out), lambda i: (0, 0)),
            pl.BlockSpec((1, d_out), lambda i: (0, 0)),
        ],
        out_specs=pl.BlockSpec((_BM, d_out), lambda i: (i, 0)),
        out_shape=jax.ShapeDtypeStruct((n, d_out), jnp.float32),
        scratch_shapes=[pltpu.VMEM((n, d_out), jnp.bfloat16)],
        compiler_params=pltpu.CompilerParams(
            dimension_semantics=("arbitrary",),
        ),
    )(adj_m, input, weight, bias2)
    return out


# speedup vs baseline: 1.0406x; 1.0147x over previous
import jax
import jax.numpy as jnp
from jax.experimental import pallas as pl
from jax.experimental.pallas import tpu as pltpu

_BM = 400


def _body(adj_ref, x_ref, w_ref, bias_ref, o_ref, sup_ref):
    i = pl.program_id(0)

    @pl.when(i == 0)
    def _():
        sup_ref[...] = jnp.dot(
            x_ref[...], w_ref[...], preferred_element_type=jnp.float32
        ).astype(jnp.bfloat16)

    a = adj_ref[...].astype(jnp.bfloat16)
    o_ref[...] = (
        jnp.dot(a, sup_ref[...], preferred_element_type=jnp.float32)
        + bias_ref[...]
    )


def kernel(input, adj_m, weight, bias):
    n, d_in = input.shape
    d_out = weight.shape[1]
    bias2 = bias.reshape(1, d_out)

    out = pl.pallas_call(
        _body,
        grid=(n // _BM,),
        in_specs=[
            pl.BlockSpec((_BM, n), lambda i: (i, 0)),
            pl.BlockSpec((n, d_in), lambda i: (0, 0)),
            pl.BlockSpec((d_in, d_out), lambda i: (0, 0)),
            pl.BlockSpec((1, d_out), lambda i: (0, 0)),
        ],
        out_specs=pl.BlockSpec((_BM, d_out), lambda i: (i, 0)),
        out_shape=jax.ShapeDtypeStruct((n, d_out), jnp.float32),
        scratch_shapes=[pltpu.VMEM((n, d_out), jnp.bfloat16)],
        compiler_params=pltpu.CompilerParams(
            dimension_semantics=("arbitrary",),
        ),
    )(adj_m, input, weight, bias2)
    return out
